# Initial kernel scaffold; baseline (speedup 1.0000x reference)
import jax
import jax.numpy as jnp
from jax.experimental import pallas as pl


def kernel(x, adapter_indices, W, b, lora_a, lora_b):
    raise NotImplementedError("write your pallas kernel here")


# speedup vs baseline: 1.0746x; 1.0746x over previous
import jax
import jax.numpy as jnp
from jax.experimental import pallas as pl
from jax.experimental.pallas import tpu as pltpu

_TM = 512


def _fused_body(idx_ref, x_ref, w_ref, bias_ref, a_ref, bb_ref, o_ref):
    x = x_ref[0]
    w = w_ref[...]
    acc = jax.lax.dot_general(
        x, w, (((1,), (1,)), ((), ())), preferred_element_type=jnp.float32)
    a = a_ref[0]
    inter = jax.lax.dot_general(
        x, a, (((1,), (1,)), ((), ())), preferred_element_type=jnp.float32)
    bb = bb_ref[0]
    lora = jax.lax.dot_general(
        inter, bb, (((1,), (1,)), ((), ())), preferred_element_type=jnp.float32)
    o_ref[0] = acc + lora + bias_ref[...]


def kernel(x, adapter_indices, W, b, lora_a, lora_b):
    B, S, DIN = x.shape
    DOUT = W.shape[0]
    E, R, _ = lora_a.shape
    idx = adapter_indices.astype(jnp.int32)
    bias = b.reshape(1, DOUT)

    grid = (B, S // _TM)

    grid_spec = pltpu.PrefetchScalarGridSpec(
        num_scalar_prefetch=1,
        grid=grid,
        in_specs=[
            pl.BlockSpec((1, _TM, DIN), lambda bi, mi, idx_ref: (bi, mi, 0)),
            pl.BlockSpec((DOUT, DIN), lambda bi, mi, idx_ref: (0, 0)),
            pl.BlockSpec((1, DOUT), lambda bi, mi, idx_ref: (0, 0)),
            pl.BlockSpec((1, R, DIN), lambda bi, mi, idx_ref: (idx_ref[bi], 0, 0)),
            pl.BlockSpec((1, DOUT, R), lambda bi, mi, idx_ref: (idx_ref[bi], 0, 0)),
        ],
        out_specs=pl.BlockSpec((1, _TM, DOUT), lambda bi, mi, idx_ref: (bi, mi, 0)),
    )

    return pl.pallas_call(
        _fused_body,
        grid_spec=grid_spec,
        out_shape=jax.ShapeDtypeStruct((B, S, DOUT), jnp.float32),
    )(idx, x, W, bias, lora_a, lora_b)
